# baseline (speedup 1.0000x reference)
import functools

import jax
import jax.numpy as jnp
from jax import lax
from jax.experimental import pallas as pl
from jax.experimental.pallas import tpu as pltpu
from jax.experimental.pallas import tpu_sc as plsc

LANES = 16


def _sc_scalar_update(action_i32, current_node_i32, current_time, used_capacity,
                      ttm_flat, tw_flat, demand_flat, B, N):
    info = plsc.get_sparse_core_info()
    NC, NS = info.num_cores, info.num_subcores
    NW = NC * NS
    assert B % NW == 0
    bpw = B // NW
    assert bpw % LANES == 0 and bpw <= 128

    mesh = plsc.VectorSubcoreMesh(core_axis_name="c", subcore_axis_name="s")

    @functools.partial(
        pl.kernel,
        out_type=(jax.ShapeDtypeStruct((B,), jnp.float32),
                  jax.ShapeDtypeStruct((B,), jnp.float32)),
        mesh=mesh,
        scratch_types=[
            pltpu.VMEM((bpw,), jnp.int32),
            pltpu.VMEM((bpw,), jnp.int32),
            pltpu.VMEM((bpw,), jnp.float32),
            pltpu.VMEM((bpw,), jnp.float32),
            pltpu.VMEM((bpw,), jnp.int32),
            pltpu.VMEM((bpw,), jnp.int32),
            pltpu.VMEM((bpw,), jnp.int32),
            pltpu.VMEM((bpw,), jnp.float32),
            pltpu.VMEM((bpw,), jnp.float32),
            pltpu.VMEM((bpw,), jnp.float32),
            pltpu.VMEM((bpw,), jnp.float32),
            pltpu.VMEM((bpw,), jnp.float32),
            pltpu.SemaphoreType.DMA,
            pltpu.SemaphoreType.DMA,
            pltpu.SemaphoreType.DMA,
        ],
    )
    def sc_kernel(act_hbm, cur_hbm, ct_hbm, uc_hbm, ttm_hbm, tw_hbm, dm_hbm,
                  sst_hbm, nl_hbm,
                  act_v, cur_v, ct_v, uc_v, ti_v, wi_v, di_v,
                  tt_v, sw_v, dmv_v, sst_v, nl_v, sem_t, sem_w, sem_d):
        wid = lax.axis_index("s") * NC + lax.axis_index("c")
        base = wid * bpw
        sl_all = pl.ds(base, bpw)
        pltpu.sync_copy(act_hbm.at[sl_all], act_v)
        pltpu.sync_copy(cur_hbm.at[sl_all], cur_v)
        pltpu.sync_copy(ct_hbm.at[sl_all], ct_v)
        pltpu.sync_copy(uc_hbm.at[sl_all], uc_v)

        for j in range(bpw // LANES):
            sl = pl.ds(j * LANES, LANES)
            a = act_v[sl]
            c = cur_v[sl]
            row = lax.iota(jnp.int32, LANES) + (base + j * LANES)
            ti_v[sl] = row * (N * N) + c * N + a
            wi_v[sl] = row * (N * 2) + a * 2
            di_v[sl] = row * N + a

        cp_t = pltpu.async_copy(ttm_hbm.at[ti_v], tt_v, sem_t)
        cp_w = pltpu.async_copy(tw_hbm.at[wi_v], sw_v, sem_w)
        cp_d = pltpu.async_copy(dm_hbm.at[di_v], dmv_v, sem_d)
        cp_t.wait()
        cp_w.wait()
        cp_d.wait()

        for j in range(bpw // LANES):
            sl = pl.ds(j * LANES, LANES)
            a = act_v[sl]
            c = cur_v[sl]
            zero = jnp.zeros((LANES,), jnp.float32)
            is_ret = (a == 0) & (c != 0)
            sst = jnp.maximum(ct_v[sl] + tt_v[sl], sw_v[sl])
            sst_v[sl] = jnp.where(is_ret, zero, sst)
            nl_v[sl] = jnp.where(is_ret, zero, uc_v[sl] + dmv_v[sl])

        pltpu.sync_copy(sst_v, sst_hbm.at[sl_all])
        pltpu.sync_copy(nl_v, nl_hbm.at[sl_all])

    return sc_kernel(action_i32, current_node_i32, current_time, used_capacity,
                     ttm_flat, tw_flat, demand_flat)


def _tc_completed_body(act_ref, comp_ref, out_ref):
    a = act_ref[...]
    comp = comp_ref[...]
    col = lax.broadcasted_iota(jnp.int32, comp.shape, 1)
    is_drop = (a % 2 == 0) & (a != 0)
    hit = (col == a) | (col == a - 1)
    out_ref[...] = comp | (is_drop & hit)


def kernel(action, current_node, current_time, used_capacity,
           travel_time_matrix, time_windows, demand, completed):
    B = action.shape[0]
    N = travel_time_matrix.shape[1]

    act_i = action.astype(jnp.int32)
    cur_i = current_node.reshape(B).astype(jnp.int32)
    ct = current_time.reshape(B)
    uc = used_capacity.reshape(B)
    ttm_flat = travel_time_matrix.reshape(B * N * N)
    tw_flat = time_windows.reshape(B * N * 2)
    dm_flat = demand.reshape(B * N)

    sst, nl = _sc_scalar_update(act_i, cur_i, ct, uc, ttm_flat, tw_flat,
                                dm_flat, B, N)

    rows = 512
    grid = B // rows
    new_completed = pl.pallas_call(
        _tc_completed_body,
        grid=(grid,),
        in_specs=[
            pl.BlockSpec((rows, 1), lambda i: (i, 0)),
            pl.BlockSpec((rows, N), lambda i: (i, 0)),
        ],
        out_specs=pl.BlockSpec((rows, N), lambda i: (i, 0)),
        out_shape=jax.ShapeDtypeStruct((B, N), jnp.bool_),
    )(act_i.reshape(B, 1), completed)

    return sst.reshape(B, 1), nl.reshape(B, 1), new_completed


# speedup vs baseline: 6.5840x; 6.5840x over previous
import functools

import jax
import jax.numpy as jnp
from jax import lax
from jax.experimental import pallas as pl
from jax.experimental.pallas import tpu as pltpu
from jax.experimental.pallas import tpu_sc as plsc

NPAD = 112


def _sc_gather(action_1d, current_1d, ttm, tw, B, N):
    info = plsc.get_sparse_core_info()
    NC, NS = info.num_cores, info.num_subcores
    NW = NC * NS
    assert B % NW == 0
    bpw = B // NW

    mesh = plsc.VectorSubcoreMesh(core_axis_name="c", subcore_axis_name="s")

    @functools.partial(
        pl.kernel,
        out_type=(jax.ShapeDtypeStruct((B, N), jnp.float32),
                  jax.ShapeDtypeStruct((B, 2), jnp.float32)),
        mesh=mesh,
        scratch_types=[
            pltpu.VMEM((bpw,), jnp.int32),
            pltpu.VMEM((bpw,), jnp.int32),
            pltpu.VMEM((bpw, N), jnp.float32),
            pltpu.VMEM((bpw, 2), jnp.float32),
            pltpu.SemaphoreType.DMA,
            pltpu.SemaphoreType.DMA,
        ],
    )
    def sc_kernel(act_hbm, cur_hbm, ttm_hbm, tw_hbm,
                  r_hbm, w_hbm,
                  act_v, cur_v, rows_v, twp_v, sem_in, sem_g):
        wid = lax.axis_index("s") * NC + lax.axis_index("c")
        base = wid * bpw
        sl_all = pl.ds(base, bpw)
        cp_a = pltpu.async_copy(act_hbm.at[sl_all], act_v, sem_in)
        cp_c = pltpu.async_copy(cur_hbm.at[sl_all], cur_v, sem_in)
        cp_a.wait()
        cp_c.wait()

        cps = []
        for j in range(bpw // 16):
            av = act_v[pl.ds(j * 16, 16)]
            cv = cur_v[pl.ds(j * 16, 16)]
            for k in range(16):
                i = j * 16 + k
                b = base + i
                a = av[k]
                c = cv[k]
                cps.append(pltpu.async_copy(ttm_hbm.at[b, c, :],
                                            rows_v.at[i], sem_g))
                cps.append(pltpu.async_copy(tw_hbm.at[b, a, :],
                                            twp_v.at[i], sem_g))
        for cp in cps:
            cp.wait()

        pltpu.sync_copy(rows_v, r_hbm.at[sl_all, :])
        pltpu.sync_copy(twp_v, w_hbm.at[sl_all, :])

    return sc_kernel(action_1d, current_1d, ttm, tw)


def _tc_body(act_ref, cur_ref, ct_ref, uc_ref, r_ref, w_ref, dm_ref, comp_ref,
             sst_ref, nl_ref, out_ref):
    a = act_ref[...]
    cur = cur_ref[...]
    comp = comp_ref[...]
    col = lax.broadcasted_iota(jnp.int32, comp.shape, 1)
    onehot = col == a
    travel = jnp.sum(jnp.where(onehot, r_ref[...], 0.0), axis=1, keepdims=True)
    sel_dm = jnp.sum(jnp.where(onehot, dm_ref[...], 0.0), axis=1, keepdims=True)
    arrival = ct_ref[...] + travel
    start_w = w_ref[...][:, 0:1]
    sst = jnp.maximum(arrival, start_w)
    is_ret = (a == 0) & (cur != 0)
    sst_ref[...] = jnp.where(is_ret, 0.0, sst)
    nl_ref[...] = jnp.where(is_ret, 0.0, uc_ref[...] + sel_dm)
    is_drop = (a % 2 == 0) & (a != 0)
    hit = onehot | (col == a - 1)
    out_ref[...] = comp | (is_drop & hit)


def kernel(action, current_node, current_time, used_capacity,
           travel_time_matrix, time_windows, demand, completed):
    B = action.shape[0]
    N = travel_time_matrix.shape[1]

    act1 = action.astype(jnp.int32)
    cur1 = current_node.reshape(B).astype(jnp.int32)

    r, w = _sc_gather(act1, cur1, travel_time_matrix, time_windows, B, N)

    rows = 512
    grid = B // rows
    col_spec = pl.BlockSpec((rows, 1), lambda i: (i, 0))
    mat_spec = pl.BlockSpec((rows, N), lambda i: (i, 0))
    sst, nl, new_completed = pl.pallas_call(
        _tc_body,
        grid=(grid,),
        in_specs=[
            col_spec,
            col_spec,
            col_spec,
            col_spec,
            mat_spec,
            pl.BlockSpec((rows, 2), lambda i: (i, 0)),
            mat_spec,
            mat_spec,
        ],
        out_specs=[col_spec, col_spec, mat_spec],
        out_shape=[
            jax.ShapeDtypeStruct((B, 1), jnp.float32),
            jax.ShapeDtypeStruct((B, 1), jnp.float32),
            jax.ShapeDtypeStruct((B, N), jnp.bool_),
        ],
    )(act1.reshape(B, 1), cur1.reshape(B, 1), current_time, used_capacity,
      r, w, demand, completed)

    return sst, nl, new_completed


# speedup vs baseline: 61.7913x; 9.3850x over previous
import functools

import jax
import jax.numpy as jnp
from jax import lax
from jax.experimental import pallas as pl
from jax.experimental.pallas import tpu as pltpu
from jax.experimental.pallas import tpu_sc as plsc


def _sc_gather_tt(action_1d, current_1d, ttm_t, B, N):
    info = plsc.get_sparse_core_info()
    NC, NS = info.num_cores, info.num_subcores
    NW = NC * NS
    assert B % NW == 0
    bpw = B // NW
    assert bpw % 16 == 0

    mesh = plsc.VectorSubcoreMesh(core_axis_name="c", subcore_axis_name="s")

    @functools.partial(
        pl.kernel,
        out_type=jax.ShapeDtypeStruct((B,), jnp.float32),
        mesh=mesh,
        scratch_types=[
            pltpu.VMEM((bpw,), jnp.int32),
            pltpu.VMEM((bpw,), jnp.int32),
            pltpu.VMEM((bpw, 16), jnp.float32),
            pltpu.VMEM((bpw,), jnp.float32),
            pltpu.SemaphoreType.DMA,
            pltpu.SemaphoreType.DMA,
        ],
    )
    def sc_kernel(act_hbm, cur_hbm, ttm_hbm, tt_hbm,
                  act_v, cur_v, win_v, tt_v, sem_in, sem_g):
        wid = lax.axis_index("s") * NC + lax.axis_index("c")
        base = wid * bpw
        sl_all = pl.ds(base, bpw)
        cp_a = pltpu.async_copy(act_hbm.at[sl_all], act_v, sem_in)
        cp_c = pltpu.async_copy(cur_hbm.at[sl_all], cur_v, sem_in)
        cp_a.wait()
        cp_c.wait()

        cps = []
        for j in range(bpw // 16):
            av = act_v[pl.ds(j * 16, 16)]
            cv = cur_v[pl.ds(j * 16, 16)]
            lanes = pl.ds(base + j * 16, 16)
            for k in range(16):
                i = j * 16 + k
                a = av[k]
                c = cv[k]
                cps.append(pltpu.async_copy(ttm_hbm.at[c, a, lanes],
                                            win_v.at[i], sem_g))
        for cp in cps:
            cp.wait()

        lane = lax.iota(jnp.int32, 16)
        for j in range(bpw // 16):
            acc = jnp.zeros((16,), jnp.float32)
            for k in range(16):
                acc = jnp.where(lane == k, win_v[pl.ds(j * 16 + k, 1), :][0],
                                acc)
            tt_v[pl.ds(j * 16, 16)] = acc

        pltpu.sync_copy(tt_v, tt_hbm.at[sl_all])

    return sc_kernel(action_1d, current_1d, ttm_t)


def _tc_body(act_ref, cur_ref, ct_ref, uc_ref, tt_ref, tw0_ref, dm_ref,
             comp_ref, sst_ref, nl_ref, out_ref):
    a = act_ref[...]
    cur = cur_ref[...]
    comp = comp_ref[...]
    row = lax.broadcasted_iota(jnp.int32, comp.shape, 0)
    onehot = row == a
    sw = jnp.sum(jnp.where(onehot, tw0_ref[...], 0.0), axis=0, keepdims=True)
    sel_dm = jnp.sum(jnp.where(onehot, dm_ref[...], 0.0), axis=0,
                     keepdims=True)
    sst = jnp.maximum(ct_ref[...] + tt_ref[...], sw)
    is_ret = (a == 0) & (cur != 0)
    sst_ref[...] = jnp.where(is_ret, 0.0, sst)
    nl_ref[...] = jnp.where(is_ret, 0.0, uc_ref[...] + sel_dm)
    is_drop = (a % 2 == 0) & (a != 0)
    hit = onehot | (row == a - 1)
    out_ref[...] = comp | (is_drop & hit)


def kernel(action, current_node, current_time, used_capacity,
           travel_time_matrix, time_windows, demand, completed):
    B = action.shape[0]
    N = travel_time_matrix.shape[1]

    act1 = action.astype(jnp.int32)
    cur1 = current_node.reshape(B).astype(jnp.int32)

    ttm_t = jnp.transpose(travel_time_matrix, (1, 2, 0))
    tw0_t = jnp.transpose(time_windows[:, :, 0], (1, 0))
    dm_t = jnp.transpose(demand, (1, 0))
    comp_t = jnp.transpose(completed, (1, 0))

    tt = _sc_gather_tt(act1, cur1, ttm_t, B, N)

    bcols = 512
    grid = B // bcols
    row_spec = pl.BlockSpec((1, bcols), lambda i: (0, i))
    mat_spec = pl.BlockSpec((N, bcols), lambda i: (0, i))
    sst_r, nl_r, comp_out_t = pl.pallas_call(
        _tc_body,
        grid=(grid,),
        in_specs=[
            row_spec,
            row_spec,
            row_spec,
            row_spec,
            row_spec,
            mat_spec,
            mat_spec,
            mat_spec,
        ],
        out_specs=[row_spec, row_spec, mat_spec],
        out_shape=[
            jax.ShapeDtypeStruct((1, B), jnp.float32),
            jax.ShapeDtypeStruct((1, B), jnp.float32),
            jax.ShapeDtypeStruct((N, B), jnp.bool_),
        ],
    )(act1.reshape(1, B), cur1.reshape(1, B), current_time.reshape(1, B),
      used_capacity.reshape(1, B), tt.reshape(1, B), tw0_t, dm_t, comp_t)

    return (sst_r.reshape(B, 1), nl_r.reshape(B, 1),
            jnp.transpose(comp_out_t, (1, 0)))


# speedup vs baseline: 68.7344x; 1.1124x over previous
import functools

import jax
import jax.numpy as jnp
from jax import lax
from jax.experimental import pallas as pl
from jax.experimental.pallas import tpu as pltpu
from jax.experimental.pallas import tpu_sc as plsc


def _sc_gather_tt(action_1d, current_1d, ttm_t, B, N):
    info = plsc.get_sparse_core_info()
    NC, NS = info.num_cores, info.num_subcores
    NW = NC * NS
    assert B % NW == 0
    bpw = B // NW
    assert bpw % 16 == 0

    mesh = plsc.VectorSubcoreMesh(core_axis_name="c", subcore_axis_name="s")

    @functools.partial(
        pl.kernel,
        out_type=jax.ShapeDtypeStruct((B,), jnp.float32),
        mesh=mesh,
        scratch_types=[
            pltpu.VMEM((bpw,), jnp.int32),
            pltpu.VMEM((bpw,), jnp.int32),
            pltpu.VMEM((bpw, 16), jnp.float32),
            pltpu.VMEM((bpw,), jnp.float32),
            pltpu.SemaphoreType.DMA,
            pltpu.SemaphoreType.DMA,
        ],
    )
    def sc_kernel(act_hbm, cur_hbm, ttm_hbm, tt_hbm,
                  act_v, cur_v, win_v, tt_v, sem_in, sem_g):
        wid = lax.axis_index("s") * NC + lax.axis_index("c")
        base = wid * bpw
        sl_all = pl.ds(base, bpw)
        cp_a = pltpu.async_copy(act_hbm.at[sl_all], act_v, sem_in)
        cp_c = pltpu.async_copy(cur_hbm.at[sl_all], cur_v, sem_in)
        cp_a.wait()
        cp_c.wait()

        cps = []
        for j in range(bpw // 16):
            av = act_v[pl.ds(j * 16, 16)]
            cv = cur_v[pl.ds(j * 16, 16)]
            lanes = pl.ds(base + j * 16, 16)
            for k in range(16):
                i = j * 16 + k
                a = av[k]
                c = cv[k]
                cps.append(pltpu.async_copy(ttm_hbm.at[c, a, lanes],
                                            win_v.at[i], sem_g))
        for cp in cps:
            cp.wait()

        lane = lax.iota(jnp.int32, 16)
        for j in range(bpw // 16):
            acc = jnp.zeros((16,), jnp.float32)
            for k in range(16):
                acc = jnp.where(lane == k, win_v[pl.ds(j * 16 + k, 1), :][0],
                                acc)
            tt_v[pl.ds(j * 16, 16)] = acc

        pltpu.sync_copy(tt_v, tt_hbm.at[sl_all])

    return sc_kernel(action_1d, current_1d, ttm_t)


def _tc_select_body(act_ref, tw0_ref, dm_ref, comp_ref,
                    sw_ref, dmsel_ref, out_ref):
    a = act_ref[...]
    comp = comp_ref[...]
    row = lax.broadcasted_iota(jnp.int32, comp.shape, 0)
    onehot = row == a
    sw_ref[...] = jnp.sum(jnp.where(onehot, tw0_ref[...], 0.0), axis=0,
                          keepdims=True)
    dmsel_ref[...] = jnp.sum(jnp.where(onehot, dm_ref[...], 0.0), axis=0,
                             keepdims=True)
    is_drop = (a % 2 == 0) & (a != 0)
    hit = onehot | (row == a - 1)
    out_ref[...] = comp | (is_drop & hit)


def _tc_math_body(act_ref, cur_ref, ct_ref, uc_ref, tt_ref, sw_ref, dm_ref,
                  sst_ref, nl_ref):
    a = act_ref[...]
    cur = cur_ref[...]
    sst = jnp.maximum(ct_ref[...] + tt_ref[...], sw_ref[...])
    is_ret = (a == 0) & (cur != 0)
    sst_ref[...] = jnp.where(is_ret, 0.0, sst)
    nl_ref[...] = jnp.where(is_ret, 0.0, uc_ref[...] + dm_ref[...])


def kernel(action, current_node, current_time, used_capacity,
           travel_time_matrix, time_windows, demand, completed):
    B = action.shape[0]
    N = travel_time_matrix.shape[1]

    act1 = action.astype(jnp.int32)
    cur1 = current_node.reshape(B).astype(jnp.int32)

    ttm_t = jnp.transpose(travel_time_matrix, (1, 2, 0))
    tw0_t = jnp.transpose(time_windows[:, :, 0], (1, 0))
    dm_t = jnp.transpose(demand, (1, 0))
    comp_t = jnp.transpose(completed, (1, 0))

    tt = _sc_gather_tt(act1, cur1, ttm_t, B, N)

    bcols = 512
    grid = B // bcols
    row_spec = pl.BlockSpec((1, bcols), lambda i: (0, i))
    mat_spec = pl.BlockSpec((N, bcols), lambda i: (0, i))
    act_row = act1.reshape(1, B)
    cur_row = cur1.reshape(1, B)
    sw_r, dm_r, comp_out_t = pl.pallas_call(
        _tc_select_body,
        grid=(grid,),
        in_specs=[
            row_spec,
            mat_spec,
            mat_spec,
            mat_spec,
        ],
        out_specs=[row_spec, row_spec, mat_spec],
        out_shape=[
            jax.ShapeDtypeStruct((1, B), jnp.float32),
            jax.ShapeDtypeStruct((1, B), jnp.float32),
            jax.ShapeDtypeStruct((N, B), jnp.bool_),
        ],
    )(act_row, tw0_t, dm_t, comp_t)

    full_row = pl.BlockSpec((1, B), lambda: (0, 0))
    sst_r, nl_r = pl.pallas_call(
        _tc_math_body,
        in_specs=[full_row] * 7,
        out_specs=[full_row, full_row],
        out_shape=[
            jax.ShapeDtypeStruct((1, B), jnp.float32),
            jax.ShapeDtypeStruct((1, B), jnp.float32),
        ],
    )(act_row, cur_row, current_time.reshape(1, B),
      used_capacity.reshape(1, B), tt.reshape(1, B), sw_r, dm_r)

    return (sst_r.reshape(B, 1), nl_r.reshape(B, 1),
            jnp.transpose(comp_out_t, (1, 0)))


# speedup vs baseline: 69.0183x; 1.0041x over previous
import functools

import jax
import jax.numpy as jnp
from jax import lax
from jax.experimental import pallas as pl
from jax.experimental.pallas import tpu as pltpu
from jax.experimental.pallas import tpu_sc as plsc


def _sc_gather_tt(action_1d, current_1d, ttm_t, B, N):
    info = plsc.get_sparse_core_info()
    NC, NS = info.num_cores, info.num_subcores
    NW = NC * NS
    assert B % NW == 0
    bpw = B // NW
    assert bpw % 16 == 0

    mesh = plsc.VectorSubcoreMesh(core_axis_name="c", subcore_axis_name="s")

    @functools.partial(
        pl.kernel,
        out_type=jax.ShapeDtypeStruct((B,), jnp.float32),
        mesh=mesh,
        compiler_params=pltpu.CompilerParams(skip_device_barrier=True),
        scratch_types=[
            pltpu.VMEM((bpw,), jnp.int32),
            pltpu.VMEM((bpw,), jnp.int32),
            pltpu.VMEM((bpw, 16), jnp.float32),
            pltpu.VMEM((bpw,), jnp.float32),
            pltpu.SemaphoreType.DMA,
            pltpu.SemaphoreType.DMA,
        ],
    )
    def sc_kernel(act_hbm, cur_hbm, ttm_hbm, tt_hbm,
                  act_v, cur_v, win_v, tt_v, sem_in, sem_g):
        wid = lax.axis_index("s") * NC + lax.axis_index("c")
        base = wid * bpw
        sl_all = pl.ds(base, bpw)
        cp_a = pltpu.async_copy(act_hbm.at[sl_all], act_v, sem_in)
        cp_c = pltpu.async_copy(cur_hbm.at[sl_all], cur_v, sem_in)
        cp_a.wait()
        cp_c.wait()

        cps = []
        for j in range(bpw // 16):
            av = act_v[pl.ds(j * 16, 16)]
            cv = cur_v[pl.ds(j * 16, 16)]
            lanes = pl.ds(base + j * 16, 16)
            for k in range(16):
                i = j * 16 + k
                a = av[k]
                c = cv[k]
                cps.append(pltpu.async_copy(ttm_hbm.at[c, a, lanes],
                                            win_v.at[i], sem_g))
        for cp in cps:
            cp.wait()

        lane = lax.iota(jnp.int32, 16)
        for j in range(bpw // 16):
            acc = jnp.zeros((16,), jnp.float32)
            for k in range(16):
                acc = jnp.where(lane == k, win_v[pl.ds(j * 16 + k, 1), :][0],
                                acc)
            tt_v[pl.ds(j * 16, 16)] = acc

        pltpu.sync_copy(tt_v, tt_hbm.at[sl_all])

    return sc_kernel(action_1d, current_1d, ttm_t)


def _tc_select_body(act_ref, tw0_ref, dm_ref, comp_ref,
                    sw_ref, dmsel_ref, out_ref):
    a = act_ref[...]
    comp = comp_ref[...]
    row = lax.broadcasted_iota(jnp.int32, comp.shape, 0)
    onehot = row == a
    sw_ref[...] = jnp.sum(jnp.where(onehot, tw0_ref[...], 0.0), axis=0,
                          keepdims=True)
    dmsel_ref[...] = jnp.sum(jnp.where(onehot, dm_ref[...], 0.0), axis=0,
                             keepdims=True)
    is_drop = (a % 2 == 0) & (a != 0)
    hit = onehot | (row == a - 1)
    out_ref[...] = comp | (is_drop & hit)


def _tc_math_body(act_ref, cur_ref, ct_ref, uc_ref, tt_ref, sw_ref, dm_ref,
                  sst_ref, nl_ref):
    a = act_ref[...]
    cur = cur_ref[...]
    sst = jnp.maximum(ct_ref[...] + tt_ref[...], sw_ref[...])
    is_ret = (a == 0) & (cur != 0)
    sst_ref[...] = jnp.where(is_ret, 0.0, sst)
    nl_ref[...] = jnp.where(is_ret, 0.0, uc_ref[...] + dm_ref[...])


def kernel(action, current_node, current_time, used_capacity,
           travel_time_matrix, time_windows, demand, completed):
    B = action.shape[0]
    N = travel_time_matrix.shape[1]

    act1 = action.astype(jnp.int32)
    cur1 = current_node.reshape(B).astype(jnp.int32)

    ttm_t = jnp.transpose(travel_time_matrix, (1, 2, 0))
    tw0_t = jnp.transpose(time_windows[:, :, 0], (1, 0))
    dm_t = jnp.transpose(demand, (1, 0))
    comp_t = jnp.transpose(completed, (1, 0))

    tt = _sc_gather_tt(act1, cur1, ttm_t, B, N)

    bcols = 512
    grid = B // bcols
    row_spec = pl.BlockSpec((1, bcols), lambda i: (0, i))
    mat_spec = pl.BlockSpec((N, bcols), lambda i: (0, i))
    act_row = act1.reshape(1, B)
    cur_row = cur1.reshape(1, B)
    sw_r, dm_r, comp_out_t = pl.pallas_call(
        _tc_select_body,
        grid=(grid,),
        in_specs=[
            row_spec,
            mat_spec,
            mat_spec,
            mat_spec,
        ],
        out_specs=[row_spec, row_spec, mat_spec],
        out_shape=[
            jax.ShapeDtypeStruct((1, B), jnp.float32),
            jax.ShapeDtypeStruct((1, B), jnp.float32),
            jax.ShapeDtypeStruct((N, B), jnp.bool_),
        ],
    )(act_row, tw0_t, dm_t, comp_t)

    full_row = pl.BlockSpec((1, B), lambda: (0, 0))
    sst_r, nl_r = pl.pallas_call(
        _tc_math_body,
        in_specs=[full_row] * 7,
        out_specs=[full_row, full_row],
        out_shape=[
            jax.ShapeDtypeStruct((1, B), jnp.float32),
            jax.ShapeDtypeStruct((1, B), jnp.float32),
        ],
    )(act_row, cur_row, current_time.reshape(1, B),
      used_capacity.reshape(1, B), tt.reshape(1, B), sw_r, dm_r)

    return (sst_r.reshape(B, 1), nl_r.reshape(B, 1),
            jnp.transpose(comp_out_t, (1, 0)))
